# baseline (speedup 1.0000x reference)
import functools

import jax
import jax.numpy as jnp
from jax import lax
from jax.experimental import pallas as pl
from jax.experimental.pallas import tpu as pltpu
from jax.experimental.pallas import tpu_sc as plsc

N = 4 * 4096 * 1024
NW = 32
PER_W = N // NW
CHUNK = 8192
NCHUNK = PER_W // CHUNK
GROUPS = CHUNK // 64


def _sc_body(yp_hbm, yt_hbm, mk_hbm, out_sum, out_cnt,
             p0, p1, t0, t1, m0, m1, stage, sem0, sem1):
    wid = lax.axis_index("s") * 2 + lax.axis_index("c")
    base = wid * PER_W

    pbufs = (p0, p1)
    tbufs = (t0, t1)
    mbufs = (m0, m1)
    sems = (sem0, sem1)

    def copies(chunk_idx, slot):
        off = pl.multiple_of(base + chunk_idx * CHUNK, CHUNK)
        return (
            pltpu.make_async_copy(yp_hbm.at[pl.ds(off, CHUNK)], pbufs[slot], sems[slot]),
            pltpu.make_async_copy(yt_hbm.at[pl.ds(off, CHUNK)], tbufs[slot], sems[slot]),
            pltpu.make_async_copy(
                mk_hbm.at[pl.ds(pl.multiple_of(off // 4, CHUNK // 4), CHUNK // 4)],
                mbufs[slot], sems[slot]),
        )

    for c in copies(0, 0):
        c.start()

    lanes = lax.iota(jnp.int32, 16)
    idxc = lanes >> 2
    shl = (3 - (lanes & 3)) << 3

    def group_body(g, carry, *, slot):
        a0, a1, a2, a3, cacc = carry
        w = mbufs[slot][pl.ds(g * 16, 16)]
        gb = g * 64
        accs = []
        for v, a in enumerate((a0, a1, a2, a3)):
            wv = jnp.take_along_axis(w, idxc + (4 * v), axis=0)
            t = wv << shl
            d = jnp.abs(pbufs[slot][pl.ds(gb + 16 * v, 16)]
                        - tbufs[slot][pl.ds(gb + 16 * v, 16)])
            accs.append(a + jnp.where(t > 0x00FFFFFF, d, 0.0))
        cacc = cacc + (w & 0x01010101)
        return accs[0], accs[1], accs[2], accs[3], cacc

    def outer(c2, carry):
        a0, a1, a2, a3, cnt = carry
        for b in (0, 1):
            cur = c2 * 2 + b
            for c in copies(cur, b):
                c.wait()
            nxt = lax.rem(cur + 1, NCHUNK)
            for c in copies(nxt, 1 - b):
                c.start()
            zero = jnp.zeros((16,), jnp.int32)
            a0, a1, a2, a3, cacc = lax.fori_loop(
                0, GROUPS, functools.partial(group_body, slot=b),
                (a0, a1, a2, a3, zero))
            cnt = (cnt + (cacc & 0xFF) + ((cacc >> 8) & 0xFF)
                   + ((cacc >> 16) & 0xFF) + ((cacc >> 24) & 0xFF))
        return a0, a1, a2, a3, cnt

    zf = jnp.zeros((16,), jnp.float32)
    a0, a1, a2, a3, cnt = lax.fori_loop(
        0, NCHUNK // 2, outer, (zf, zf, zf, zf, jnp.zeros((16,), jnp.int32)))

    for c in copies(0, 0):
        c.wait()

    stage[pl.ds(0, 16)] = (a0 + a1) + (a2 + a3)
    stage[pl.ds(16, 16)] = cnt.astype(jnp.float32)
    pltpu.sync_copy(stage.at[pl.ds(0, 16)], out_sum.at[pl.ds(wid * 16, 16)])
    pltpu.sync_copy(stage.at[pl.ds(16, 16)], out_cnt.at[pl.ds(wid * 16, 16)])


_sc_call = functools.partial(
    pl.kernel,
    out_type=[
        jax.ShapeDtypeStruct((NW * 16,), jnp.float32),
        jax.ShapeDtypeStruct((NW * 16,), jnp.float32),
    ],
    mesh=plsc.VectorSubcoreMesh(core_axis_name="c", subcore_axis_name="s"),
    scratch_types=[
        pltpu.VMEM((CHUNK,), jnp.float32),
        pltpu.VMEM((CHUNK,), jnp.float32),
        pltpu.VMEM((CHUNK,), jnp.float32),
        pltpu.VMEM((CHUNK,), jnp.float32),
        pltpu.VMEM((CHUNK // 4,), jnp.int32),
        pltpu.VMEM((CHUNK // 4,), jnp.int32),
        pltpu.VMEM((32,), jnp.float32),
        pltpu.SemaphoreType.DMA,
        pltpu.SemaphoreType.DMA,
    ],
)(_sc_body)


def kernel(y_pred, y_true, mask):
    yp = y_pred.reshape(-1)
    yt = y_true.reshape(-1)
    mk = mask.reshape(-1).view(jnp.int8).view(jnp.int32)
    sums, cnts = _sc_call(yp, yt, mk)
    return jnp.sqrt(jnp.sum(sums) / jnp.sum(cnts))


# speedup vs baseline: 9.2318x; 9.2318x over previous
import functools

import jax
import jax.numpy as jnp
from jax import lax
from jax.experimental import pallas as pl
from jax.experimental.pallas import tpu as pltpu
from jax.experimental.pallas import tpu_sc as plsc

ROWS = 16384
COLS = 1024
WCOLS = COLS // 4
NW = 32
ROWS_W = ROWS // NW
BR = 8
NBLK = ROWS_W // BR
CGR = COLS // 64


def _sc_body(yp_hbm, yt_hbm, mk_hbm, out_sum, out_cnt,
             p0, p1, t0, t1, m0, m1, stage, sem0, sem1):
    wid = lax.axis_index("s") * 2 + lax.axis_index("c")
    base = wid * ROWS_W

    pbufs = (p0, p1)
    tbufs = (t0, t1)
    mbufs = (m0, m1)
    sems = (sem0, sem1)

    def copies(blk_idx, slot):
        r0 = pl.multiple_of(base + blk_idx * BR, BR)
        return (
            pltpu.make_async_copy(yp_hbm.at[pl.ds(r0, BR), :], pbufs[slot], sems[slot]),
            pltpu.make_async_copy(yt_hbm.at[pl.ds(r0, BR), :], tbufs[slot], sems[slot]),
            pltpu.make_async_copy(mk_hbm.at[pl.ds(r0, BR), :], mbufs[slot], sems[slot]),
        )

    for c in copies(0, 0):
        c.start()

    lanes = lax.iota(jnp.int32, 16)
    idxc = lanes >> 2
    shl = (3 - (lanes & 3)) << 3

    def make_group_body(slot, r):
        def group_body(g, carry):
            a0, a1, a2, a3, cacc = carry
            w = mbufs[slot][r, pl.ds(g * 16, 16)]
            gb = g * 64
            accs = []
            for v, a in enumerate((a0, a1, a2, a3)):
                wv = jnp.take_along_axis(w, idxc + (4 * v), axis=0)
                t = wv << shl
                d = jnp.abs(pbufs[slot][r, pl.ds(gb + 16 * v, 16)]
                            - tbufs[slot][r, pl.ds(gb + 16 * v, 16)])
                accs.append(a + jnp.where(t > 0x00FFFFFF, d, 0.0))
            cacc = cacc + (w & 0x01010101)
            return accs[0], accs[1], accs[2], accs[3], cacc
        return group_body

    def outer(c2, carry):
        a0, a1, a2, a3, cnt = carry
        for b in (0, 1):
            cur = c2 * 2 + b
            for c in copies(cur, b):
                c.wait()
            nxt = lax.rem(cur + 1, NBLK)
            for c in copies(nxt, 1 - b):
                c.start()
            cacc = jnp.zeros((16,), jnp.int32)
            for r in range(BR):
                a0, a1, a2, a3, cacc = lax.fori_loop(
                    0, CGR, make_group_body(b, r), (a0, a1, a2, a3, cacc))
            cnt = (cnt + (cacc & 0xFF) + ((cacc >> 8) & 0xFF)
                   + ((cacc >> 16) & 0xFF) + ((cacc >> 24) & 0xFF))
        return a0, a1, a2, a3, cnt

    zf = jnp.zeros((16,), jnp.float32)
    a0, a1, a2, a3, cnt = lax.fori_loop(
        0, NBLK // 2, outer, (zf, zf, zf, zf, jnp.zeros((16,), jnp.int32)))

    for c in copies(0, 0):
        c.wait()

    stage[pl.ds(0, 16)] = (a0 + a1) + (a2 + a3)
    stage[pl.ds(16, 16)] = cnt.astype(jnp.float32)
    pltpu.sync_copy(stage.at[pl.ds(0, 16)], out_sum.at[pl.ds(wid * 16, 16)])
    pltpu.sync_copy(stage.at[pl.ds(16, 16)], out_cnt.at[pl.ds(wid * 16, 16)])


_sc_call = functools.partial(
    pl.kernel,
    out_type=[
        jax.ShapeDtypeStruct((NW * 16,), jnp.float32),
        jax.ShapeDtypeStruct((NW * 16,), jnp.float32),
    ],
    mesh=plsc.VectorSubcoreMesh(core_axis_name="c", subcore_axis_name="s"),
    scratch_types=[
        pltpu.VMEM((BR, COLS), jnp.float32),
        pltpu.VMEM((BR, COLS), jnp.float32),
        pltpu.VMEM((BR, COLS), jnp.float32),
        pltpu.VMEM((BR, COLS), jnp.float32),
        pltpu.VMEM((BR, WCOLS), jnp.int32),
        pltpu.VMEM((BR, WCOLS), jnp.int32),
        pltpu.VMEM((32,), jnp.float32),
        pltpu.SemaphoreType.DMA,
        pltpu.SemaphoreType.DMA,
    ],
)(_sc_body)


def kernel(y_pred, y_true, mask):
    yp = y_pred.reshape(ROWS, COLS)
    yt = y_true.reshape(ROWS, COLS)
    mk = mask.reshape(ROWS, COLS).view(jnp.int8).view(jnp.int32)
    sums, cnts = _sc_call(yp, yt, mk)
    return jnp.sqrt(jnp.sum(sums) / jnp.sum(cnts))
